# baseline (speedup 1.0000x reference)
import functools

import jax
import jax.numpy as jnp
from jax import lax
from jax.experimental import pallas as pl
from jax.experimental.pallas import tpu as pltpu
from jax.experimental.pallas import tpu_sc as plsc

_NC = 2
_NS = 16
_NW = _NC * _NS
_LANES = 16
_CHUNK = 128


@functools.lru_cache(maxsize=None)
def _make_sc_gather(B):
    per_w = B // _NW
    n_chunk = per_w // _CHUNK
    mesh = plsc.VectorSubcoreMesh(core_axis_name="c", subcore_axis_name="s")

    @functools.partial(
        pl.kernel,
        out_type=jax.ShapeDtypeStruct((B, 2), jnp.float32),
        mesh=mesh,
        compiler_params=pltpu.CompilerParams(use_tc_tiling_on_sc=False),
        scratch_types=[
            pltpu.VMEM((n_chunk, _CHUNK), jnp.int32),
            pltpu.VMEM((per_w, 2), jnp.float32),
            pltpu.SemaphoreType.DMA,
        ],
    )
    def sc_gather(idx_hbm, conf_hbm, gout_hbm, idx_v, rows_v, sem):
        wid = lax.axis_index("s") * _NC + lax.axis_index("c")
        base = wid * per_w
        pltpu.sync_copy(idx_hbm.at[wid], idx_v)
        copies = []
        for k in range(n_chunk):
            copies.append(pltpu.async_copy(
                conf_hbm.at[idx_v.at[k]],
                rows_v.at[pl.ds(k * _CHUNK, _CHUNK)], sem))
        for c in copies:
            c.wait()
        pltpu.sync_copy(rows_v, gout_hbm.at[pl.ds(base, per_w), :])

    return sc_gather


@functools.lru_cache(maxsize=None)
def _make_tc_loss(B):
    def body(cf_ref, o2_ref, out_ref):
        cf = cf_ref[...]
        o = o2_ref[...][:, 0]
        x = 2.0 * o - 1.0
        sp = jnp.log1p(jnp.exp(x))
        term = sp * (cf[:, 0] + cf[:, 1]) - x * cf[:, 1]
        out_ref[0, 0] = jnp.sum(term) * (1.0 / B)

    return pl.pallas_call(
        body,
        out_shape=jax.ShapeDtypeStruct((1, 1), jnp.float32),
        in_specs=[
            pl.BlockSpec(memory_space=pltpu.VMEM),
            pl.BlockSpec(memory_space=pltpu.VMEM),
        ],
        out_specs=pl.BlockSpec(memory_space=pltpu.SMEM),
    )


def kernel(outputs, patch_index, confidence):
    B = outputs.shape[0]
    per_w = B // _NW
    rows = (2 * B) // 128
    idx = patch_index.reshape((_NW, per_w // _CHUNK, _CHUNK))
    gathered = _make_sc_gather(B)(idx, confidence)
    return _make_tc_loss(B)(gathered, outputs)[0, 0]


# speedup vs baseline: 24.7170x; 24.7170x over previous
import functools

import jax
import jax.numpy as jnp
from jax import lax
from jax.experimental import pallas as pl
from jax.experimental.pallas import tpu as pltpu
from jax.experimental.pallas import tpu_sc as plsc

_NC = 2
_NS = 16
_NW = _NC * _NS
_LANES = 16
_CHUNK = 128

_SP_C0 = 0.693147186409334
_SP_C1 = 0.1249997313784969
_SP_C2 = -5.206379217398428e-03
_SP_C3 = 3.4224919293833467e-04
_SP_C4 = -2.109280949471386e-05


@functools.lru_cache(maxsize=None)
def _make_sc_partials(B):
    per_w = B // _NW
    n_chunk = per_w // _CHUNK
    n_vec = per_w // _LANES
    mesh = plsc.VectorSubcoreMesh(core_axis_name="c", subcore_axis_name="s")

    @functools.partial(
        pl.kernel,
        out_type=jax.ShapeDtypeStruct((_NW, _LANES), jnp.float32),
        mesh=mesh,
        scratch_types=[
            pltpu.VMEM((n_chunk, _CHUNK), jnp.int32),
            pltpu.VMEM((per_w,), jnp.float32),
            pltpu.VMEM((per_w,), jnp.float32),
            pltpu.VMEM((per_w,), jnp.float32),
            pltpu.VMEM((_LANES,), jnp.float32),
            pltpu.SemaphoreType.DMA,
        ],
    )
    def sc_partials(o_hbm, idx_hbm, c0_hbm, c1_hbm, out_hbm,
                    idx_v, c0_v, c1_v, o_v, part_v, sem):
        wid = lax.axis_index("s") * _NC + lax.axis_index("c")
        base = wid * per_w
        pltpu.sync_copy(idx_hbm.at[wid], idx_v)
        copies = []
        for k in range(n_chunk):
            sl = pl.ds(k * _CHUNK, _CHUNK)
            copies.append(pltpu.async_copy(
                c0_hbm.at[idx_v.at[k]], c0_v.at[sl], sem))
            copies.append(pltpu.async_copy(
                c1_hbm.at[idx_v.at[k]], c1_v.at[sl], sem))
        pltpu.sync_copy(o_hbm.at[pl.ds(base, per_w)], o_v)
        for c in copies:
            c.wait()

        def body(i, acc):
            sl = pl.ds(i * _LANES, _LANES)
            o = o_v[sl]
            c0 = c0_v[sl]
            c1 = c1_v[sl]
            x = 2.0 * o - 1.0
            u = x * x
            sp = 0.5 * x + (_SP_C0 + u * (_SP_C1 + u * (
                _SP_C2 + u * (_SP_C3 + u * _SP_C4))))
            return acc + (sp * (c0 + c1) - x * c1)

        acc = lax.fori_loop(0, n_vec, body, jnp.zeros((_LANES,), jnp.float32))
        part_v[...] = acc
        pltpu.sync_copy(part_v, out_hbm.at[wid])

    return sc_partials


@functools.lru_cache(maxsize=None)
def _make_reduce(B):
    def body(p_ref, o_ref):
        o_ref[0, 0] = jnp.sum(p_ref[...]) * (1.0 / B)

    return pl.pallas_call(
        body,
        out_shape=jax.ShapeDtypeStruct((1, 1), jnp.float32),
        in_specs=[pl.BlockSpec(memory_space=pltpu.VMEM)],
        out_specs=pl.BlockSpec(memory_space=pltpu.SMEM),
    )


def kernel(outputs, patch_index, confidence):
    B = outputs.shape[0]
    per_w = B // _NW
    o_flat = outputs.reshape((B,))
    c0 = confidence[:, 0]
    c1 = confidence[:, 1]
    idx = patch_index.reshape((_NW, per_w // _CHUNK, _CHUNK))
    partials = _make_sc_partials(B)(o_flat, idx, c0, c1)
    return _make_reduce(B)(partials)[0, 0]
